# baseline (speedup 1.0000x reference)
import functools
import math

import jax
import jax.numpy as jnp
from jax import lax
from jax.experimental import pallas as pl
from jax.experimental.pallas import tpu as pltpu
from jax.experimental.pallas import tpu_sc as plsc

D = 32
V = 1000000
SCALE = math.sqrt(32.0)
NC, NS = 2, 16
NW = NC * NS
SEQ = 200
ROWS = 4096
RT = ROWS // 128
GRP = 8
NGRP = SEQ // GRP

TCOLS = 1024
TGRID = (V + TCOLS - 1) // TCOLS

_mesh = plsc.VectorSubcoreMesh(
    core_axis_name="c", subcore_axis_name="s", num_cores=NC, num_subcores=NS
)


def _tcx_body(x_ref, o_ref):
    xt = (x_ref[...] * SCALE).T
    xt3 = xt.reshape(TCOLS // 4, 4, D)
    o_ref[...] = jnp.concatenate(
        [xt3[:, j, :] for j in range(4)], axis=1
    )


_scale_transpose = pl.pallas_call(
    _tcx_body,
    grid=(TGRID,),
    in_specs=[pl.BlockSpec((D, TCOLS), lambda i: (0, i))],
    out_specs=pl.BlockSpec((TCOLS // 4, 128), lambda i: (i, 0)),
    out_shape=jax.ShapeDtypeStruct((V // 4, 128), jnp.float32),
)


def _k2_body(table_hbm, tok_hbm, out_hbm, idx2_v, rows_v, tiles_v, sem):
    w = lax.axis_index("s") * NC + lax.axis_index("c")
    lane = lax.iota(jnp.int32, 16)
    fbs = [lax.shift_right_logical(lane + 16 * h, 3) for h in range(2)]
    fis = [lax.bitwise_and(lane + 16 * h, 7) for h in range(2)]

    pltpu.sync_copy(tok_hbm.at[:, pl.ds(w * 128, 128)], idx2_v)

    def group(g, carry):
        c0 = g * GRP
        cps = [
            pltpu.async_copy(
                table_hbm.at[idx2_v.at[c0 + b]], rows_v.at[b], sem
            )
            for b in range(GRP)
        ]
        for b in range(GRP):
            cps[b].wait()
            tb = tiles_v.at[b]

            def assemble(r4, carry2):
                for j in range(4):
                    r = r4 * 4 + j
                    rv = jnp.full((16,), r, jnp.int32)
                    for h in range(2):
                        vals = rows_v[b, r, pl.ds(16 * h, 16)]
                        plsc.store_scatter(tb, [fbs[h], fis[h], rv], vals)
                return carry2

            lax.fori_loop(0, 32, assemble, 0)
        pltpu.sync_copy(
            tiles_v.at[:, :, :, pl.ds(0, 128)],
            out_hbm.at[pl.ds(c0, GRP), :, w],
        )
        return carry

    lax.fori_loop(0, NGRP, group, 0)


_emb_lookup = pl.kernel(
    _k2_body,
    out_type=jax.ShapeDtypeStruct((SEQ, 4, RT, 8, 128), jnp.float32),
    mesh=_mesh,
    compiler_params=pltpu.CompilerParams(
        use_tc_tiling_on_sc=False, needs_layout_passes=False
    ),
    scratch_types=[
        pltpu.VMEM((SEQ, 128), jnp.int32),
        pltpu.VMEM((GRP, 128, D), jnp.float32),
        pltpu.VMEM((GRP, 4, 8, 133), jnp.float32),
        pltpu.SemaphoreType.DMA,
    ],
)


@jax.jit
def kernel(tokens, table):
    tok_t = tokens.T.astype(jnp.int32)
    tab_r = _scale_transpose(table.T).reshape(V, D)
    out5 = _emb_lookup(tab_r, tok_t)
    return out5.transpose(2, 4, 0, 1, 3).reshape(ROWS, SEQ, D)


# speedup vs baseline: 1.2497x; 1.2497x over previous
import functools
import math

import jax
import jax.numpy as jnp
from jax import lax
from jax.experimental import pallas as pl
from jax.experimental.pallas import tpu as pltpu
from jax.experimental.pallas import tpu_sc as plsc

D = 32
SCALE = math.sqrt(32.0)
NC, NS = 2, 16
NW = NC * NS
SEQ = 200
ROWS = 4096
RT = ROWS // 128
GRP = 5
NGRP = SEQ // GRP

_mesh = plsc.VectorSubcoreMesh(
    core_axis_name="c", subcore_axis_name="s", num_cores=NC, num_subcores=NS
)


def _k2_body(
    table_hbm, tok_hbm, out_hbm,
    idx2_v, rows_a, rows_b, tiles_a, tiles_b,
    gsem_a, gsem_b, osem_a, osem_b,
):
    w = lax.axis_index("s") * NC + lax.axis_index("c")
    lane = lax.iota(jnp.int32, 16)
    fbs = [lax.shift_right_logical(lane + 16 * h, 3) for h in range(2)]
    fis = [lax.bitwise_and(lane + 16 * h, 7) for h in range(2)]
    sets = [(rows_a, tiles_a, gsem_a, osem_a), (rows_b, tiles_b, gsem_b, osem_b)]

    pltpu.sync_copy(tok_hbm.at[:, pl.ds(w * 128, 128)], idx2_v)

    def fire(g, rows, gsem):
        c0 = g * GRP
        for b in range(GRP):
            pltpu.async_copy(table_hbm.at[idx2_v.at[c0 + b]], rows.at[b], gsem)

    fire(0, rows_a, gsem_a)

    def pair(gg, carry):
        for p in range(2):
            rows, tiles, gsem, osem = sets[p]
            nrows, _, ngsem, _ = sets[1 - p]
            g = gg * 2 + p

            @pl.when(g + 1 < NGRP)
            def _prefetch():
                fire(g + 1, nrows, ngsem)

            for b in range(GRP):
                pltpu.make_async_copy(
                    table_hbm.at[pl.ds(0, 128)], rows.at[b], gsem
                ).wait()

            @pl.when(g >= 2)
            def _outwait():
                pltpu.make_async_copy(
                    out_hbm.at[pl.ds(0, GRP), :, 0],
                    tiles.at[:, :, :, pl.ds(0, 128)],
                    osem,
                ).wait()

            for b in range(GRP):
                tb = tiles.at[b]

                def assemble(r4, carry2):
                    for j in range(4):
                        r = r4 * 4 + j
                        rv = jnp.full((16,), r, jnp.int32)
                        for h in range(2):
                            vals = rows[b, r, pl.ds(16 * h, 16)] * SCALE
                            plsc.store_scatter(
                                tb, [fbs[h], fis[h], rv], vals
                            )
                    return carry2

                lax.fori_loop(0, 32, assemble, 0)

            pltpu.async_copy(
                tiles.at[:, :, :, pl.ds(0, 128)],
                out_hbm.at[pl.ds(g * GRP, GRP), :, w],
                osem,
            )
        return carry

    lax.fori_loop(0, NGRP // 2, pair, 0)
    for p in range(2):
        rows, tiles, gsem, osem = sets[p]
        pltpu.make_async_copy(
            out_hbm.at[pl.ds(0, GRP), :, 0],
            tiles.at[:, :, :, pl.ds(0, 128)],
            osem,
        ).wait()


_emb_lookup = pl.kernel(
    _k2_body,
    out_type=jax.ShapeDtypeStruct((SEQ, 4, RT, 8, 128), jnp.float32),
    mesh=_mesh,
    compiler_params=pltpu.CompilerParams(
        use_tc_tiling_on_sc=False, needs_layout_passes=False
    ),
    scratch_types=[
        pltpu.VMEM((SEQ, 128), jnp.int32),
        pltpu.VMEM((GRP, 128, D), jnp.float32),
        pltpu.VMEM((GRP, 128, D), jnp.float32),
        pltpu.VMEM((GRP, 4, 8, 133), jnp.float32),
        pltpu.VMEM((GRP, 4, 8, 133), jnp.float32),
        pltpu.SemaphoreType.DMA,
        pltpu.SemaphoreType.DMA,
        pltpu.SemaphoreType.DMA,
        pltpu.SemaphoreType.DMA,
    ],
)


@jax.jit
def kernel(tokens, table):
    tok_t = tokens.T.astype(jnp.int32)
    out5 = _emb_lookup(table, tok_t)
    return out5.transpose(2, 4, 0, 1, 3).reshape(ROWS, SEQ, D)
